# baseline (speedup 1.0000x reference)
import functools
import jax
import jax.numpy as jnp
from jax import lax
from jax.experimental import pallas as pl
from jax.experimental.pallas import tpu as pltpu
from jax.experimental.pallas import tpu_sc as plsc

B, S, T, NG, E, N, R, D, NB = 512, 8, 8, 2000, 4000, 10000, 100, 128, 8
NGP = 2048
EP = 4096
HROWS = T * NGP
EROWS = T * EP
NC, NS = 2, 16
NW = NC * NS
HALF = HROWS // NC


_CH = 128


def _gather_pipe(tasks, bufs, gsems, wsems):
  n = len(tasks)
  gd = [None, None]
  wd = [None, None]
  for i, (tab, idx, _, _) in enumerate(tasks):
    p = i & 1
    if i >= 2:
      wd[p].wait()
    gd[p] = pltpu.async_copy(tab.at[idx], bufs[p], gsems[p])
    if i >= 1:
      _, _, out_prev, off_prev = tasks[i - 1]
      gd[1 - p].wait()
      wd[1 - p] = pltpu.async_copy(
          bufs[1 - p], out_prev.at[pl.ds(off_prev, _CH)], wsems[1 - p])
  p = (n - 1) & 1
  _, _, out_last, off_last = tasks[n - 1]
  gd[p].wait()
  wd[p] = pltpu.async_copy(
      bufs[p], out_last.at[pl.ds(off_last, _CH)], wsems[p])
  if n >= 2:
    wd[1 - p].wait()
  wd[p].wait()


def _sc_gather(table, idx):
  m = idx.shape[0]
  n = m // (NW * _CH)
  mesh = plsc.VectorSubcoreMesh(core_axis_name="c", subcore_axis_name="s")

  @functools.partial(
      pl.kernel, mesh=mesh,
      out_type=jax.ShapeDtypeStruct((m, D), jnp.float32),
      scratch_types=[
          pltpu.VMEM((n * _CH,), jnp.int32),
          pltpu.VMEM((_CH, D), jnp.float32),
          pltpu.VMEM((_CH, D), jnp.float32),
          pltpu.SemaphoreType.DMA,
          pltpu.SemaphoreType.DMA,
          pltpu.SemaphoreType.DMA,
          pltpu.SemaphoreType.DMA,
      ],
  )
  def k(table_hbm, idx_hbm, out_hbm, idx_all, buf0, buf1, g0, g1, w0, w1):
    wid = lax.axis_index("s") * NC + lax.axis_index("c")
    base = wid * (n * _CH)
    pltpu.sync_copy(idx_hbm.at[pl.ds(base, n * _CH)], idx_all)
    tasks = [(table_hbm, idx_all.at[pl.ds(i * _CH, _CH)], out_hbm,
              base + i * _CH) for i in range(n)]
    _gather_pipe(tasks, (buf0, buf1), (g0, g1), (w0, w1))

  return k(table, idx)


def _sc_gather_final(t0, i0, t1, i1, t2, i2):
  m = B * S
  mesh = plsc.VectorSubcoreMesh(core_axis_name="c", subcore_axis_name="s")

  @functools.partial(
      pl.kernel, mesh=mesh,
      out_type=jax.ShapeDtypeStruct((3 * m, D), jnp.float32),
      scratch_types=[
          pltpu.VMEM((3 * _CH,), jnp.int32),
          pltpu.VMEM((_CH, D), jnp.float32),
          pltpu.VMEM((_CH, D), jnp.float32),
          pltpu.SemaphoreType.DMA,
          pltpu.SemaphoreType.DMA,
          pltpu.SemaphoreType.DMA,
          pltpu.SemaphoreType.DMA,
      ],
  )
  def k(t0_h, i0_h, t1_h, i1_h, t2_h, i2_h, out_h, idx_all,
        buf0, buf1, g0, g1, w0, w1):
    wid = lax.axis_index("s") * NC + lax.axis_index("c")
    off = wid * _CH
    tabs = (t0_h, t1_h, t2_h)
    for j, ih in enumerate((i0_h, i1_h, i2_h)):
      pltpu.sync_copy(ih.at[pl.ds(off, _CH)],
                      idx_all.at[pl.ds(j * _CH, _CH)])
    tasks = [(tabs[j], idx_all.at[pl.ds(j * _CH, _CH)], out_h,
              j * m + off) for j in range(3)]
    _gather_pipe(tasks, (buf0, buf1), (g0, g1), (w0, w1))

  return k(t0, i0, t1, i1, t2, i2)


def _sc_scatter_add(vals, dst_local, width):
  slots_sc = EROWS // NC
  slots_w = slots_sc // NS
  ch = 128
  n_chunks = slots_w // ch
  out_rows_w = HALF // NS
  mesh = plsc.VectorSubcoreMesh(core_axis_name="c", subcore_axis_name="s")
  zeros = jnp.zeros((out_rows_w, width), jnp.float32)

  @functools.partial(
      pl.kernel, mesh=mesh,
      out_type=jax.ShapeDtypeStruct((HROWS, width), jnp.float32),
      scratch_types=[
          pltpu.VMEM((n_chunks, ch), jnp.int32),
          pltpu.VMEM((ch, width), jnp.float32),
          pltpu.VMEM((ch, width), jnp.float32),
          pltpu.VMEM_SHARED((HALF, width), jnp.float32),
          pltpu.SemaphoreType.DMA,
          pltpu.SemaphoreType.DMA,
          pltpu.SemaphoreType.DMA,
          pltpu.SemaphoreType.DMA,
      ],
  )
  def k(vals_hbm, dstl_hbm, zeros_hbm, out_hbm, idx_m, buf0, buf1, acc_sh,
        l0, l1, s0, s1):
    cid = lax.axis_index("c")
    sid = lax.axis_index("s")
    pltpu.sync_copy(zeros_hbm, acc_sh.at[pl.ds(sid * out_rows_w, out_rows_w)])
    crow = cid * (slots_sc // ch) + sid * (slots_w // ch)
    pltpu.sync_copy(dstl_hbm.at[pl.ds(crow, n_chunks)], idx_m)
    plsc.subcore_barrier()

    bufs = (buf0, buf1)
    lsems = (l0, l1)
    ssems = (s0, s1)
    ld = [None, None]
    sd = [None, None]
    for i in range(n_chunks):
      p = i & 1
      if i >= 2:
        sd[p].wait()
      ld[p] = pltpu.async_copy(
          vals_hbm.at[pl.ds((crow + i) * ch, ch)], bufs[p], lsems[p])
      if i >= 1:
        ld[1 - p].wait()
        sd[1 - p] = pltpu.async_copy(
            bufs[1 - p], acc_sh.at[idx_m.at[i - 1]], ssems[1 - p], add=True)
    p = (n_chunks - 1) & 1
    ld[p].wait()
    sd[p] = pltpu.async_copy(
        bufs[p], acc_sh.at[idx_m.at[n_chunks - 1]], ssems[p], add=True)
    if n_chunks >= 2:
      sd[1 - p].wait()
    sd[p].wait()

    plsc.subcore_barrier()
    pltpu.sync_copy(
        acc_sh.at[pl.ds(sid * out_rows_w, out_rows_w)],
        out_hbm.at[pl.ds(cid * HALF + sid * out_rows_w, out_rows_w)])

  return k(vals, dst_local.reshape(EROWS // ch, ch), zeros)


_BLKE = 4096
_BLKH = 2048
_BLKB = 64


def _msg_body(hs_ref, typ_ref, compx_ref, vcat_ref, out_ref):
  typ = typ_ref[:, :]
  onehot = (typ == lax.broadcasted_iota(jnp.int32, (_BLKE, 128), 1)
            ).astype(jnp.bfloat16)
  cfull = jnp.dot(onehot, compx_ref[:, :],
                  preferred_element_type=jnp.float32)
  hsv = jnp.dot(hs_ref[:, :].astype(jnp.bfloat16), vcat_ref[:, :],
                preferred_element_type=jnp.float32)
  prod = hsv * cfull
  acc = prod[:, 0:D]
  for b in range(1, NB):
    acc = acc + prod[:, b * D:(b + 1) * D]
  out_ref[:, :] = acc


def _tc_messages(hs, typ2, comp_pad, vcat):
  grid = (EROWS // _BLKE,)
  return pl.pallas_call(
      _msg_body,
      grid=grid,
      in_specs=[
          pl.BlockSpec((_BLKE, D), lambda i: (i, 0)),
          pl.BlockSpec((_BLKE, 1), lambda i: (i, 0)),
          pl.BlockSpec((128, NB * D), lambda i: (0, 0)),
          pl.BlockSpec((D, NB * D), lambda i: (0, 0)),
      ],
      out_specs=pl.BlockSpec((_BLKE, D), lambda i: (i, 0)),
      out_shape=jax.ShapeDtypeStruct((EROWS, D), jnp.float32),
  )(hs, typ2, comp_pad, vcat)


def _deg_body(dst_ref, out_ref):
  j = pl.program_id(1)
  nodes = j * 512 + lax.broadcasted_iota(jnp.int32, (512, 1), 0)
  acc = jnp.zeros((512, 1), jnp.float32)
  for ec in range(EP // 512):
    chunk = dst_ref[0, ec, :].reshape(1, 512)
    acc = acc + jnp.sum((nodes == chunk).astype(jnp.float32),
                        axis=1, keepdims=True)
  out_ref[:, :] = acc


def _tc_degrees(dst3):
  return pl.pallas_call(
      _deg_body,
      grid=(T, NGP // 512),
      in_specs=[pl.BlockSpec((1, EP // 512, 512), lambda t, j: (t, 0, 0))],
      out_specs=pl.BlockSpec((512, 1), lambda t, j: (t * (NGP // 512) + j, 0)),
      out_shape=jax.ShapeDtypeStruct((HROWS, 1), jnp.float32),
  )(dst3)


def _update_body(agg_ref, deg_ref, h_ref, w_ref, out_ref):
  r = 1.0 / jnp.maximum(deg_ref[:, :], 1.0)
  hw = jnp.dot(h_ref[:, :], w_ref[:, :], preferred_element_type=jnp.float32)
  out_ref[:, :] = jnp.maximum(agg_ref[:, :] * r + hw, 0.0)


def _tc_update(agg, deg2, h, wself):
  grid = (HROWS // _BLKH,)
  return pl.pallas_call(
      _update_body,
      grid=grid,
      in_specs=[
          pl.BlockSpec((_BLKH, D), lambda i: (i, 0)),
          pl.BlockSpec((_BLKH, 1), lambda i: (i, 0)),
          pl.BlockSpec((_BLKH, D), lambda i: (i, 0)),
          pl.BlockSpec((D, D), lambda i: (0, 0)),
      ],
      out_specs=pl.BlockSpec((_BLKH, D), lambda i: (i, 0)),
      out_shape=jax.ShapeDtypeStruct((HROWS, D), jnp.float32),
  )(agg, deg2, h, wself)


def _assemble_body(rgcn_ref, ent_ref, glob_ref, rel_ref, len_ref, out_ref):
  rows = _BLKB * S
  pos = lax.broadcasted_iota(jnp.int32, (rows, 1), 0) % S
  m = (pos < len_ref[:, :]).astype(jnp.float32)
  mean_rel = jnp.sum(rel_ref[:, :], axis=0, keepdims=True) * (1.0 / R)
  out_ref[:, 0 * D:1 * D] = rgcn_ref[:, :] * m
  out_ref[:, 1 * D:2 * D] = ent_ref[:, :] * m
  out_ref[:, 2 * D:3 * D] = jnp.broadcast_to(mean_rel, (rows, D)) * m
  out_ref[:, 3 * D:4 * D] = glob_ref[:, :] * m


def _tc_assemble(rgcn_e, ent_e, glob_e, rel_pad, len2):
  grid = (B // _BLKB,)
  rows = _BLKB * S
  return pl.pallas_call(
      _assemble_body,
      grid=grid,
      in_specs=[
          pl.BlockSpec((rows, D), lambda i: (i, 0)),
          pl.BlockSpec((rows, D), lambda i: (i, 0)),
          pl.BlockSpec((rows, D), lambda i: (i, 0)),
          pl.BlockSpec((128, D), lambda i: (0, 0)),
          pl.BlockSpec((rows, 1), lambda i: (i, 0)),
      ],
      out_specs=pl.BlockSpec((rows, 4 * D), lambda i: (i, 0)),
      out_shape=jax.ShapeDtypeStruct((B * S, 4 * D), jnp.float32),
  )(rgcn_e, ent_e, glob_e, rel_pad, len2)


def kernel(entity_ids, hist_t, hist_len, entity_pos, node_ids, edge_src,
           edge_dst, edge_type, entity_embeds, rel_embeds, global_emb,
           V1, comp1, Wself1, V2, comp2, Wself2):
  i32 = jnp.int32
  tvec = jnp.arange(T, dtype=i32)

  node_pad = jnp.zeros((T, NGP), i32).at[:, :NG].set(node_ids.astype(i32))
  node_flat = node_pad.reshape(HROWS)

  pad_cols = jnp.full((T, EP - E), NGP - 1, i32)
  src_g = jnp.concatenate([edge_src.astype(i32), pad_cols], axis=1)
  src_flat = (src_g + tvec[:, None] * NGP).reshape(EROWS)
  dst_g = jnp.concatenate([edge_dst.astype(i32), pad_cols], axis=1)
  dst_local = (dst_g + (tvec[:, None] % (T // NC)) * NGP).reshape(EROWS)
  typ = jnp.concatenate(
      [edge_type.astype(i32), jnp.full((T, EP - E), R, i32)], axis=1)
  typ2 = typ.reshape(EROWS, 1)

  def _compx(comp):
    cp = jnp.zeros((128, NB), jnp.float32).at[:R].set(comp)
    return jnp.repeat(cp, D, axis=1).astype(jnp.bfloat16)

  comp1_pad = _compx(comp1)
  comp2_pad = _compx(comp2)
  vcat1 = V1.transpose(1, 0, 2).reshape(D, NB * D).astype(jnp.bfloat16)
  vcat2 = V2.transpose(1, 0, 2).reshape(D, NB * D).astype(jnp.bfloat16)
  rel_pad = jnp.zeros((128, D), jnp.float32).at[:R].set(rel_embeds)

  dst3 = dst_g.reshape(T, EP // 512, 512)

  hist_t32 = hist_t.astype(i32)
  rgcn_idx = (hist_t32 * NGP + entity_pos.astype(i32)).reshape(B * S)
  glob_idx = hist_t32.reshape(B * S)
  ent_idx = jnp.repeat(entity_ids.astype(i32), S)
  len2 = jnp.repeat(hist_len.astype(i32), S).reshape(B * S, 1)

  deg2 = _tc_degrees(dst3)

  h = _sc_gather(entity_embeds, node_flat)

  for comp_pad, vcat, wself in ((comp1_pad, vcat1, Wself1),
                                (comp2_pad, vcat2, Wself2)):
    hs = _sc_gather(h, src_flat)
    msg = _tc_messages(hs, typ2, comp_pad, vcat)
    agg = _sc_scatter_add(msg, dst_local, D)
    h = _tc_update(agg, deg2, h, wself)

  fin = _sc_gather_final(h, rgcn_idx, entity_embeds, ent_idx,
                         global_emb, glob_idx)
  rgcn_e = fin[:B * S]
  ent_e = fin[B * S:2 * B * S]
  glob_e = fin[2 * B * S:]
  seq = _tc_assemble(rgcn_e, ent_e, glob_e, rel_pad, len2)
  return seq.reshape(B, S, 4 * D), hist_len


# speedup vs baseline: 1.0285x; 1.0285x over previous
import functools
import jax
import jax.numpy as jnp
from jax import lax
from jax.experimental import pallas as pl
from jax.experimental.pallas import tpu as pltpu
from jax.experimental.pallas import tpu_sc as plsc

B, S, T, NG, E, N, R, D, NB = 512, 8, 8, 2000, 4000, 10000, 100, 128, 8
NGP = 2048
EP = 4096
HROWS = T * NGP
EROWS = T * EP
NC, NS = 2, 16
NW = NC * NS
HALF = HROWS // NC


_CH = 128


def _gather_pipe(tasks, bufs, gsems, wsems):
  n = len(tasks)
  gd = [None, None]
  wd = [None, None]
  for i, (tab, idx, _, _) in enumerate(tasks):
    p = i & 1
    if i >= 2:
      wd[p].wait()
    gd[p] = pltpu.async_copy(tab.at[idx], bufs[p], gsems[p])
    if i >= 1:
      _, _, out_prev, off_prev = tasks[i - 1]
      gd[1 - p].wait()
      wd[1 - p] = pltpu.async_copy(
          bufs[1 - p], out_prev.at[pl.ds(off_prev, _CH)], wsems[1 - p])
  p = (n - 1) & 1
  _, _, out_last, off_last = tasks[n - 1]
  gd[p].wait()
  wd[p] = pltpu.async_copy(
      bufs[p], out_last.at[pl.ds(off_last, _CH)], wsems[p])
  if n >= 2:
    wd[1 - p].wait()
  wd[p].wait()


def _sc_gather(table, idx):
  m = idx.shape[0]
  n = m // (NW * _CH)
  mesh = plsc.VectorSubcoreMesh(core_axis_name="c", subcore_axis_name="s")

  @functools.partial(
      pl.kernel, mesh=mesh,
      out_type=jax.ShapeDtypeStruct((m, D), jnp.float32),
      scratch_types=[
          pltpu.VMEM((n * _CH,), jnp.int32),
          pltpu.VMEM((_CH, D), jnp.float32),
          pltpu.VMEM((_CH, D), jnp.float32),
          pltpu.SemaphoreType.DMA,
          pltpu.SemaphoreType.DMA,
          pltpu.SemaphoreType.DMA,
          pltpu.SemaphoreType.DMA,
      ],
  )
  def k(table_hbm, idx_hbm, out_hbm, idx_all, buf0, buf1, g0, g1, w0, w1):
    wid = lax.axis_index("s") * NC + lax.axis_index("c")
    base = wid * (n * _CH)
    pltpu.sync_copy(idx_hbm.at[pl.ds(base, n * _CH)], idx_all)
    tasks = [(table_hbm, idx_all.at[pl.ds(i * _CH, _CH)], out_hbm,
              base + i * _CH) for i in range(n)]
    _gather_pipe(tasks, (buf0, buf1), (g0, g1), (w0, w1))

  return k(table, idx)


def _sc_gather_init(ent_table, glob_table, big_idx):
  mrows = HROWS + 2 * B * S
  nh = HROWS // (NW * _CH)
  nt = nh + 2
  mesh = plsc.VectorSubcoreMesh(core_axis_name="c", subcore_axis_name="s")

  @functools.partial(
      pl.kernel, mesh=mesh,
      out_type=jax.ShapeDtypeStruct((mrows, D), jnp.float32),
      scratch_types=[
          pltpu.VMEM((nt * _CH,), jnp.int32),
          pltpu.VMEM((_CH, D), jnp.float32),
          pltpu.VMEM((_CH, D), jnp.float32),
          pltpu.SemaphoreType.DMA,
          pltpu.SemaphoreType.DMA,
          pltpu.SemaphoreType.DMA,
          pltpu.SemaphoreType.DMA,
      ],
  )
  def k(ent_h, glob_h, idx_h, out_h, idx_all, buf0, buf1, g0, g1, w0, w1):
    wid = lax.axis_index("s") * NC + lax.axis_index("c")
    hbase = wid * (nh * _CH)
    ebase = HROWS + wid * _CH
    gbase = HROWS + B * S + wid * _CH
    pltpu.sync_copy(idx_h.at[pl.ds(hbase, nh * _CH)],
                    idx_all.at[pl.ds(0, nh * _CH)])
    pltpu.sync_copy(idx_h.at[pl.ds(ebase, _CH)],
                    idx_all.at[pl.ds(nh * _CH, _CH)])
    pltpu.sync_copy(idx_h.at[pl.ds(gbase, _CH)],
                    idx_all.at[pl.ds((nh + 1) * _CH, _CH)])
    tasks = [(ent_h, idx_all.at[pl.ds(i * _CH, _CH)], out_h,
              hbase + i * _CH) for i in range(nh)]
    tasks.append((ent_h, idx_all.at[pl.ds(nh * _CH, _CH)], out_h, ebase))
    tasks.append((glob_h, idx_all.at[pl.ds((nh + 1) * _CH, _CH)], out_h,
                  gbase))
    _gather_pipe(tasks, (buf0, buf1), (g0, g1), (w0, w1))

  return k(ent_table, glob_table, big_idx)


def _sc_scatter_add(vals, dst_local, width):
  slots_sc = EROWS // NC
  slots_w = slots_sc // NS
  ch = 128
  n_chunks = slots_w // ch
  out_rows_w = HALF // NS
  mesh = plsc.VectorSubcoreMesh(core_axis_name="c", subcore_axis_name="s")
  zeros = jnp.zeros((out_rows_w, width), jnp.float32)

  @functools.partial(
      pl.kernel, mesh=mesh,
      out_type=jax.ShapeDtypeStruct((HROWS, width), jnp.float32),
      scratch_types=[
          pltpu.VMEM((n_chunks, ch), jnp.int32),
          pltpu.VMEM((ch, width), jnp.float32),
          pltpu.VMEM((ch, width), jnp.float32),
          pltpu.VMEM_SHARED((HALF, width), jnp.float32),
          pltpu.SemaphoreType.DMA,
          pltpu.SemaphoreType.DMA,
          pltpu.SemaphoreType.DMA,
          pltpu.SemaphoreType.DMA,
      ],
  )
  def k(vals_hbm, dstl_hbm, zeros_hbm, out_hbm, idx_m, buf0, buf1, acc_sh,
        l0, l1, s0, s1):
    cid = lax.axis_index("c")
    sid = lax.axis_index("s")
    pltpu.sync_copy(zeros_hbm, acc_sh.at[pl.ds(sid * out_rows_w, out_rows_w)])
    crow = cid * (slots_sc // ch) + sid * (slots_w // ch)
    pltpu.sync_copy(dstl_hbm.at[pl.ds(crow, n_chunks)], idx_m)
    plsc.subcore_barrier()

    bufs = (buf0, buf1)
    lsems = (l0, l1)
    ssems = (s0, s1)
    ld = [None, None]
    sd = [None, None]
    for i in range(n_chunks):
      p = i & 1
      if i >= 2:
        sd[p].wait()
      ld[p] = pltpu.async_copy(
          vals_hbm.at[pl.ds((crow + i) * ch, ch)], bufs[p], lsems[p])
      if i >= 1:
        ld[1 - p].wait()
        sd[1 - p] = pltpu.async_copy(
            bufs[1 - p], acc_sh.at[idx_m.at[i - 1]], ssems[1 - p], add=True)
    p = (n_chunks - 1) & 1
    ld[p].wait()
    sd[p] = pltpu.async_copy(
        bufs[p], acc_sh.at[idx_m.at[n_chunks - 1]], ssems[p], add=True)
    if n_chunks >= 2:
      sd[1 - p].wait()
    sd[p].wait()

    plsc.subcore_barrier()
    pltpu.sync_copy(
        acc_sh.at[pl.ds(sid * out_rows_w, out_rows_w)],
        out_hbm.at[pl.ds(cid * HALF + sid * out_rows_w, out_rows_w)])

  return k(vals, dst_local.reshape(EROWS // ch, ch), zeros)


_BLKE = 4096
_BLKH = 2048
_BLKB = 64


def _msg_body(hs_ref, typ_ref, compx_ref, vcat_ref, out_ref):
  typ = typ_ref[:, :]
  onehot = (typ == lax.broadcasted_iota(jnp.int32, (_BLKE, 128), 1)
            ).astype(jnp.bfloat16)
  cfull = jnp.dot(onehot, compx_ref[:, :],
                  preferred_element_type=jnp.float32)
  hsv = jnp.dot(hs_ref[:, :].astype(jnp.bfloat16), vcat_ref[:, :],
                preferred_element_type=jnp.float32)
  prod = hsv * cfull
  acc = prod[:, 0:D]
  for b in range(1, NB):
    acc = acc + prod[:, b * D:(b + 1) * D]
  out_ref[:, :] = acc


def _tc_messages(hs, typ2, comp_pad, vcat):
  grid = (EROWS // _BLKE,)
  return pl.pallas_call(
      _msg_body,
      grid=grid,
      in_specs=[
          pl.BlockSpec((_BLKE, D), lambda i: (i, 0)),
          pl.BlockSpec((_BLKE, 1), lambda i: (i, 0)),
          pl.BlockSpec((128, NB * D), lambda i: (0, 0)),
          pl.BlockSpec((D, NB * D), lambda i: (0, 0)),
      ],
      out_specs=pl.BlockSpec((_BLKE, D), lambda i: (i, 0)),
      out_shape=jax.ShapeDtypeStruct((EROWS, D), jnp.float32),
  )(hs, typ2, comp_pad, vcat)


def _deg_body(dst_ref, out_ref):
  j = pl.program_id(1)
  nodes = j * 512 + lax.broadcasted_iota(jnp.int32, (512, 1), 0)
  acc = jnp.zeros((512, 1), jnp.float32)
  for ec in range(EP // 512):
    chunk = dst_ref[0, ec, :].reshape(1, 512)
    acc = acc + jnp.sum((nodes == chunk).astype(jnp.float32),
                        axis=1, keepdims=True)
  out_ref[:, :] = acc


def _tc_degrees(dst3):
  return pl.pallas_call(
      _deg_body,
      grid=(T, NGP // 512),
      in_specs=[pl.BlockSpec((1, EP // 512, 512), lambda t, j: (t, 0, 0))],
      out_specs=pl.BlockSpec((512, 1), lambda t, j: (t * (NGP // 512) + j, 0)),
      out_shape=jax.ShapeDtypeStruct((HROWS, 1), jnp.float32),
  )(dst3)


def _update_body(agg_ref, deg_ref, h_ref, w_ref, out_ref):
  r = 1.0 / jnp.maximum(deg_ref[:, :], 1.0)
  hw = jnp.dot(h_ref[:, :], w_ref[:, :], preferred_element_type=jnp.float32)
  out_ref[:, :] = jnp.maximum(agg_ref[:, :] * r + hw, 0.0)


def _tc_update(agg, deg2, h, wself):
  grid = (HROWS // _BLKH,)
  return pl.pallas_call(
      _update_body,
      grid=grid,
      in_specs=[
          pl.BlockSpec((_BLKH, D), lambda i: (i, 0)),
          pl.BlockSpec((_BLKH, 1), lambda i: (i, 0)),
          pl.BlockSpec((_BLKH, D), lambda i: (i, 0)),
          pl.BlockSpec((D, D), lambda i: (0, 0)),
      ],
      out_specs=pl.BlockSpec((_BLKH, D), lambda i: (i, 0)),
      out_shape=jax.ShapeDtypeStruct((HROWS, D), jnp.float32),
  )(agg, deg2, h, wself)


def _assemble_body(rgcn_ref, ent_ref, glob_ref, rel_ref, len_ref, out_ref):
  rows = _BLKB * S
  pos = lax.broadcasted_iota(jnp.int32, (rows, 1), 0) % S
  m = (pos < len_ref[:, :]).astype(jnp.float32)
  mean_rel = jnp.sum(rel_ref[:, :], axis=0, keepdims=True) * (1.0 / R)
  out_ref[:, 0 * D:1 * D] = rgcn_ref[:, :] * m
  out_ref[:, 1 * D:2 * D] = ent_ref[:, :] * m
  out_ref[:, 2 * D:3 * D] = jnp.broadcast_to(mean_rel, (rows, D)) * m
  out_ref[:, 3 * D:4 * D] = glob_ref[:, :] * m


def _tc_assemble(rgcn_e, ent_e, glob_e, rel_pad, len2):
  grid = (B // _BLKB,)
  rows = _BLKB * S
  return pl.pallas_call(
      _assemble_body,
      grid=grid,
      in_specs=[
          pl.BlockSpec((rows, D), lambda i: (i, 0)),
          pl.BlockSpec((rows, D), lambda i: (i, 0)),
          pl.BlockSpec((rows, D), lambda i: (i, 0)),
          pl.BlockSpec((128, D), lambda i: (0, 0)),
          pl.BlockSpec((rows, 1), lambda i: (i, 0)),
      ],
      out_specs=pl.BlockSpec((rows, 4 * D), lambda i: (i, 0)),
      out_shape=jax.ShapeDtypeStruct((B * S, 4 * D), jnp.float32),
  )(rgcn_e, ent_e, glob_e, rel_pad, len2)


def kernel(entity_ids, hist_t, hist_len, entity_pos, node_ids, edge_src,
           edge_dst, edge_type, entity_embeds, rel_embeds, global_emb,
           V1, comp1, Wself1, V2, comp2, Wself2):
  i32 = jnp.int32
  tvec = jnp.arange(T, dtype=i32)

  node_pad = jnp.zeros((T, NGP), i32).at[:, :NG].set(node_ids.astype(i32))
  node_flat = node_pad.reshape(HROWS)

  pad_cols = jnp.full((T, EP - E), NGP - 1, i32)
  src_g = jnp.concatenate([edge_src.astype(i32), pad_cols], axis=1)
  src_flat = (src_g + tvec[:, None] * NGP).reshape(EROWS)
  dst_g = jnp.concatenate([edge_dst.astype(i32), pad_cols], axis=1)
  dst_local = (dst_g + (tvec[:, None] % (T // NC)) * NGP).reshape(EROWS)
  typ = jnp.concatenate(
      [edge_type.astype(i32), jnp.full((T, EP - E), R, i32)], axis=1)
  typ2 = typ.reshape(EROWS, 1)

  def _compx(comp):
    cp = jnp.zeros((128, NB), jnp.float32).at[:R].set(comp)
    return jnp.repeat(cp, D, axis=1).astype(jnp.bfloat16)

  comp1_pad = _compx(comp1)
  comp2_pad = _compx(comp2)
  vcat1 = V1.transpose(1, 0, 2).reshape(D, NB * D).astype(jnp.bfloat16)
  vcat2 = V2.transpose(1, 0, 2).reshape(D, NB * D).astype(jnp.bfloat16)
  rel_pad = jnp.zeros((128, D), jnp.float32).at[:R].set(rel_embeds)

  dst3 = dst_g.reshape(T, EP // 512, 512)

  hist_t32 = hist_t.astype(i32)
  rgcn_idx = (hist_t32 * NGP + entity_pos.astype(i32)).reshape(B * S)
  glob_idx = hist_t32.reshape(B * S)
  ent_idx = jnp.repeat(entity_ids.astype(i32), S)
  len2 = jnp.repeat(hist_len.astype(i32), S).reshape(B * S, 1)

  deg2 = _tc_degrees(dst3)

  big_idx = jnp.concatenate([node_flat, ent_idx, glob_idx])
  big = _sc_gather_init(entity_embeds, global_emb, big_idx)
  h = big[:HROWS]
  ent_e = big[HROWS:HROWS + B * S]
  glob_e = big[HROWS + B * S:]

  for comp_pad, vcat, wself in ((comp1_pad, vcat1, Wself1),
                                (comp2_pad, vcat2, Wself2)):
    hs = _sc_gather(h, src_flat)
    msg = _tc_messages(hs, typ2, comp_pad, vcat)
    agg = _sc_scatter_add(msg, dst_local, D)
    h = _tc_update(agg, deg2, h, wself)

  rgcn_e = _sc_gather(h, rgcn_idx)
  seq = _tc_assemble(rgcn_e, ent_e, glob_e, rel_pad, len2)
  return seq.reshape(B, S, 4 * D), hist_len
